# baseline (speedup 1.0000x reference)
import functools

import jax
import jax.numpy as jnp
from jax import lax
from jax.experimental import pallas as pl
from jax.experimental.pallas import tpu as pltpu
from jax.experimental.pallas import tpu_sc as plsc

NUM_E = 16
D = 1024
F = 2048
K = 2
T = 2048
S = T * K
B = 256
NB_MAX = S // B + NUM_E
NPAD = NB_MAX * B


_GTB = 256


def _gate_body(x_ref, gw_ref, gb_ref, topi_ref, g_ref, rank_ref, cnt_ref,
               carry_ref):
    i = pl.program_id(0)
    logits = jnp.dot(x_ref[...], gw_ref[...],
                     preferred_element_type=jnp.float32) + gb_ref[0][None, :]
    idx16 = lax.broadcasted_iota(jnp.int32, logits.shape, 1)
    m1 = jnp.max(logits, axis=1, keepdims=True)
    i1 = jnp.min(jnp.where(logits == m1, idx16, 9999), axis=1, keepdims=True)
    masked = jnp.where(idx16 == i1, -1e30, logits)
    m2 = jnp.max(masked, axis=1, keepdims=True)
    i2 = jnp.min(jnp.where(masked == m2, idx16, 9999), axis=1, keepdims=True)
    e2 = jnp.exp(m2 - m1)
    g1 = 1.0 / (1.0 + e2)
    topi_ref[:, 0:1] = i1
    topi_ref[:, 1:2] = i2
    g_ref[:, 0:1] = g1
    g_ref[:, 1:2] = 1.0 - g1

    @pl.when(i == 0)
    def _():
        carry_ref[...] = jnp.zeros((1, NUM_E), jnp.float32)

    oh0 = (idx16 == i1).astype(jnp.float32)
    oh1 = (idx16 == i2).astype(jnp.float32)
    both = oh0 + oh1
    r_i = lax.broadcasted_iota(jnp.int32, (_GTB, _GTB), 0)
    c_i = lax.broadcasted_iota(jnp.int32, (_GTB, _GTB), 1)
    tril = (c_i < r_i).astype(jnp.float32)
    pre = jnp.dot(tril, both, preferred_element_type=jnp.float32)
    base = pre + carry_ref[0][None, :]
    rank_ref[:, 0:1] = jnp.sum(oh0 * base, axis=1, keepdims=True
                               ).astype(jnp.int32)
    rank_ref[:, 1:2] = jnp.sum(oh1 * base, axis=1, keepdims=True
                               ).astype(jnp.int32)
    new_carry = carry_ref[...] + jnp.sum(both, axis=0, keepdims=True)
    carry_ref[...] = new_carry
    cnt_ref[...] = new_carry.astype(jnp.int32)


def _gate(x, gate_w, gate_b):
    return pl.pallas_call(
        _gate_body,
        grid=(T // _GTB,),
        in_specs=[
            pl.BlockSpec((_GTB, D), lambda i: (i, 0)),
            pl.BlockSpec((D, NUM_E), lambda i: (0, 0)),
            pl.BlockSpec((1, NUM_E), lambda i: (0, 0)),
        ],
        out_specs=[
            pl.BlockSpec((_GTB, K), lambda i: (i, 0)),
            pl.BlockSpec((_GTB, K), lambda i: (i, 0)),
            pl.BlockSpec((_GTB, K), lambda i: (i, 0)),
            pl.BlockSpec((1, NUM_E), lambda i: (0, 0)),
        ],
        out_shape=[
            jax.ShapeDtypeStruct((T, K), jnp.int32),
            jax.ShapeDtypeStruct((T, K), jnp.float32),
            jax.ShapeDtypeStruct((T, K), jnp.int32),
            jax.ShapeDtypeStruct((1, NUM_E), jnp.int32),
        ],
        scratch_shapes=[pltpu.VMEM((1, NUM_E), jnp.float32)],
    )(x, gate_w, gate_b.reshape(1, NUM_E))


def _routing(topi, rank, counts):
    e_flat = topi.reshape(-1)
    nb_e = (counts + B - 1) // B
    cum_nb = jnp.cumsum(nb_e)
    nb = cum_nb[-1].astype(jnp.int32)
    padded_off = jnp.concatenate(
        [jnp.zeros(1, jnp.int32), cum_nb[:-1].astype(jnp.int32)]) * B
    dest_pos = padded_off[e_flat] + rank.reshape(-1)
    arange_s = jnp.arange(S, dtype=jnp.int32)
    be = jnp.searchsorted(cum_nb, jnp.minimum(jnp.arange(NB_MAX), nb - 1),
                          side="right").astype(jnp.int32)
    dest_slot = (S + jnp.arange(NPAD, dtype=jnp.int32) % (NPAD - S)
                 ).at[dest_pos].set(arange_s, mode="drop",
                                    unique_indices=True)
    dp = dest_pos.reshape(T, K)
    return dp[:, 0], dp[:, 1], be, dest_slot, nb


_GW = 32
_GC = 64
_GPW = NPAD // _GW
_TPW = T // _GW


def _sc_mesh():
    return plsc.VectorSubcoreMesh(core_axis_name="c", subcore_axis_name="s")


def _sc_dispatch_body(x_hbm, pos0_hbm, pos1_hbm, out_hbm, idx0_v, idx1_v,
                      rows_v, isem, wsem):
    wid = lax.axis_index("s") * 2 + lax.axis_index("c")
    i0 = pltpu.async_copy(pos0_hbm.at[wid], idx0_v, isem)
    i1 = pltpu.async_copy(pos1_hbm.at[wid], idx1_v, isem)
    pltpu.sync_copy(x_hbm.at[pl.ds(wid * _TPW, _TPW)], rows_v)
    i0.wait()
    i1.wait()
    w0 = pltpu.async_copy(rows_v, out_hbm.at[idx0_v], wsem)
    w1 = pltpu.async_copy(rows_v, out_hbm.at[idx1_v], wsem)
    w0.wait()
    w1.wait()


def _sc_dispatch(x, pos0, pos1):
    run = functools.partial(
        pl.kernel,
        mesh=_sc_mesh(),
        out_type=jax.ShapeDtypeStruct((NPAD, D), jnp.float32),
        scratch_types=[
            pltpu.VMEM((_TPW,), jnp.int32),
            pltpu.VMEM((_TPW,), jnp.int32),
            pltpu.VMEM((_TPW, D), jnp.float32),
            pltpu.SemaphoreType.DMA,
            pltpu.SemaphoreType.DMA,
        ],
    )(_sc_dispatch_body)
    return run(x, pos0.reshape(_GW, _TPW), pos1.reshape(_GW, _TPW))


def _sc_scatter_body(ys_hbm, didx_hbm, yu_hbm, idx_v, rows_v, sem):
    wid = lax.axis_index("s") * 2 + lax.axis_index("c")
    base = wid * _GPW
    for c in range(_GPW // _GC):
        o = base + c * _GC
        pltpu.sync_copy(didx_hbm.at[pl.ds(o, _GC)], idx_v)
        pltpu.sync_copy(ys_hbm.at[pl.ds(o, _GC)], rows_v)
        pltpu.async_copy(rows_v, yu_hbm.at[idx_v], sem).wait()


def _sc_scatter(ys, dest_slot, nb):
    del nb
    run = functools.partial(
        pl.kernel,
        mesh=_sc_mesh(),
        out_type=jax.ShapeDtypeStruct((NPAD, D), jnp.float32),
        scratch_types=[
            pltpu.VMEM((_GC,), jnp.int32),
            pltpu.VMEM((_GC, D), jnp.float32),
            pltpu.SemaphoreType.DMA,
        ],
    )(_sc_scatter_body)
    return run(ys, dest_slot)


def _ffn_body(be_ref, xb_ref, nb_ref, xs_ref, w1_ref, b1_ref, w2_ref, b2_ref,
              out_ref):
    bidx = pl.program_id(0)

    @pl.when(bidx < nb_ref[0])
    def _():
        h = jnp.dot(xs_ref[...], w1_ref[0],
                    preferred_element_type=jnp.float32) + b1_ref[0, 0][None, :]
        h = jnp.maximum(h, 0.0)
        y = jnp.dot(h, w2_ref[0],
                    preferred_element_type=jnp.float32) + b2_ref[0, 0][None, :]
        out_ref[...] = y


def _ffn(xs, w1, b1, w2, b2, be, nb):
    xb = jnp.minimum(jnp.arange(NB_MAX, dtype=jnp.int32), nb - 1)
    grid_spec = pltpu.PrefetchScalarGridSpec(
        num_scalar_prefetch=3,
        grid=(NB_MAX,),
        in_specs=[
            pl.BlockSpec((B, D), lambda b, be, xb, nbv: (xb[b], 0)),
            pl.BlockSpec((1, D, F), lambda b, be, xb, nbv: (be[b], 0, 0)),
            pl.BlockSpec((1, 1, F), lambda b, be, xb, nbv: (be[b], 0, 0)),
            pl.BlockSpec((1, F, D), lambda b, be, xb, nbv: (be[b], 0, 0)),
            pl.BlockSpec((1, 1, D), lambda b, be, xb, nbv: (be[b], 0, 0)),
        ],
        out_specs=pl.BlockSpec((B, D), lambda b, be, xb, nbv: (xb[b], 0)),
    )
    return pl.pallas_call(
        _ffn_body,
        grid_spec=grid_spec,
        out_shape=jax.ShapeDtypeStruct((NPAD, D), jnp.float32),
        compiler_params=pltpu.CompilerParams(
            dimension_semantics=("arbitrary",)),
    )(be, xb, nb.reshape(1), xs, w1, b1.reshape(NUM_E, 1, F), w2,
      b2.reshape(NUM_E, 1, D))


def _combine_body(yu_ref, g_ref, out_ref):
    y3 = yu_ref[...].reshape(yu_ref.shape[0] // K, K, D)
    out_ref[...] = (g_ref[:, 0:1] * y3[:, 0, :] + g_ref[:, 1:2] * y3[:, 1, :])


def _combine(yu, g):
    TB = 512
    return pl.pallas_call(
        _combine_body,
        grid=(T // TB,),
        in_specs=[
            pl.BlockSpec((TB * K, D), lambda i: (i, 0)),
            pl.BlockSpec((TB, K), lambda i: (i, 0)),
        ],
        out_specs=pl.BlockSpec((TB, D), lambda i: (i, 0)),
        out_shape=jax.ShapeDtypeStruct((T, D), jnp.float32),
    )(yu, g)


def kernel(moe_inp, gate_w, gate_b, w1, b1, w2, b2):
    topi, g, rank, counts = _gate(moe_inp, gate_w, gate_b)
    pos0, pos1, be, dest_slot, nb = _routing(topi, rank, counts[0])
    xs = _sc_dispatch(moe_inp, pos0, pos1)
    ys = _ffn(xs, w1, b1, w2, b2, be, nb)
    yu = _sc_scatter(ys, dest_slot, nb)
    return _combine(yu, g)


# speedup vs baseline: 1.0139x; 1.0139x over previous
import functools

import jax
import jax.numpy as jnp
from jax import lax
from jax.experimental import pallas as pl
from jax.experimental.pallas import tpu as pltpu
from jax.experimental.pallas import tpu_sc as plsc

NUM_E = 16
D = 1024
F = 2048
K = 2
T = 2048
S = T * K
B = 256
NB_MAX = S // B + NUM_E
NPAD = NB_MAX * B


_GTB = 512


def _gate_body(x_ref, gw_ref, gb_ref, topi_ref, g_ref, rank_ref, cnt_ref,
               carry_ref):
    i = pl.program_id(0)
    logits = jnp.dot(x_ref[...], gw_ref[...],
                     preferred_element_type=jnp.float32) + gb_ref[0][None, :]
    idx16 = lax.broadcasted_iota(jnp.int32, logits.shape, 1)
    m1 = jnp.max(logits, axis=1, keepdims=True)
    i1 = jnp.min(jnp.where(logits == m1, idx16, 9999), axis=1, keepdims=True)
    masked = jnp.where(idx16 == i1, -1e30, logits)
    m2 = jnp.max(masked, axis=1, keepdims=True)
    i2 = jnp.min(jnp.where(masked == m2, idx16, 9999), axis=1, keepdims=True)
    e2 = jnp.exp(m2 - m1)
    g1 = 1.0 / (1.0 + e2)
    topi_ref[:, 0:1] = i1
    topi_ref[:, 1:2] = i2
    g_ref[:, 0:1] = g1
    g_ref[:, 1:2] = 1.0 - g1

    @pl.when(i == 0)
    def _():
        carry_ref[...] = jnp.zeros((1, NUM_E), jnp.float32)

    oh0 = (idx16 == i1).astype(jnp.float32)
    oh1 = (idx16 == i2).astype(jnp.float32)
    both = oh0 + oh1
    r_i = lax.broadcasted_iota(jnp.int32, (_GTB, _GTB), 0)
    c_i = lax.broadcasted_iota(jnp.int32, (_GTB, _GTB), 1)
    tril = (c_i < r_i).astype(jnp.float32)
    pre = jnp.dot(tril, both, preferred_element_type=jnp.float32)
    base = pre + carry_ref[0][None, :]
    rank_ref[:, 0:1] = jnp.sum(oh0 * base, axis=1, keepdims=True
                               ).astype(jnp.int32)
    rank_ref[:, 1:2] = jnp.sum(oh1 * base, axis=1, keepdims=True
                               ).astype(jnp.int32)
    new_carry = carry_ref[...] + jnp.sum(both, axis=0, keepdims=True)
    carry_ref[...] = new_carry
    cnt_ref[...] = new_carry.astype(jnp.int32)


def _gate(x, gate_w, gate_b):
    return pl.pallas_call(
        _gate_body,
        grid=(T // _GTB,),
        in_specs=[
            pl.BlockSpec((_GTB, D), lambda i: (i, 0)),
            pl.BlockSpec((D, NUM_E), lambda i: (0, 0)),
            pl.BlockSpec((1, NUM_E), lambda i: (0, 0)),
        ],
        out_specs=[
            pl.BlockSpec((_GTB, K), lambda i: (i, 0)),
            pl.BlockSpec((_GTB, K), lambda i: (i, 0)),
            pl.BlockSpec((_GTB, K), lambda i: (i, 0)),
            pl.BlockSpec((1, NUM_E), lambda i: (0, 0)),
        ],
        out_shape=[
            jax.ShapeDtypeStruct((T, K), jnp.int32),
            jax.ShapeDtypeStruct((T, K), jnp.float32),
            jax.ShapeDtypeStruct((T, K), jnp.int32),
            jax.ShapeDtypeStruct((1, NUM_E), jnp.int32),
        ],
        scratch_shapes=[pltpu.VMEM((1, NUM_E), jnp.float32)],
    )(x, gate_w, gate_b.reshape(1, NUM_E))


def _routing(topi, rank, counts):
    e_flat = topi.reshape(-1)
    nb_e = (counts + B - 1) // B
    cum_nb = jnp.cumsum(nb_e)
    nb = cum_nb[-1].astype(jnp.int32)
    padded_off = jnp.concatenate(
        [jnp.zeros(1, jnp.int32), cum_nb[:-1].astype(jnp.int32)]) * B
    dest_pos = padded_off[e_flat] + rank.reshape(-1)
    arange_s = jnp.arange(S, dtype=jnp.int32)
    be = jnp.searchsorted(cum_nb, jnp.minimum(jnp.arange(NB_MAX), nb - 1),
                          side="right").astype(jnp.int32)
    dest_slot = (S + jnp.arange(NPAD, dtype=jnp.int32) % (NPAD - S)
                 ).at[dest_pos].set(arange_s, mode="drop",
                                    unique_indices=True)
    dp = dest_pos.reshape(T, K)
    return dp[:, 0], dp[:, 1], be, dest_slot, nb


_GW = 32
_GC = 64
_GPW = NPAD // _GW
_TPW = T // _GW


def _sc_mesh():
    return plsc.VectorSubcoreMesh(core_axis_name="c", subcore_axis_name="s")


def _sc_dispatch_body(x_hbm, pos0_hbm, pos1_hbm, out_hbm, idx0_v, idx1_v,
                      rows_v, isem, wsem):
    wid = lax.axis_index("s") * 2 + lax.axis_index("c")
    i0 = pltpu.async_copy(pos0_hbm.at[wid], idx0_v, isem)
    i1 = pltpu.async_copy(pos1_hbm.at[wid], idx1_v, isem)
    pltpu.sync_copy(x_hbm.at[pl.ds(wid * _TPW, _TPW)], rows_v)
    i0.wait()
    i1.wait()
    w0 = pltpu.async_copy(rows_v, out_hbm.at[idx0_v], wsem)
    w1 = pltpu.async_copy(rows_v, out_hbm.at[idx1_v], wsem)
    w0.wait()
    w1.wait()


def _sc_dispatch(x, pos0, pos1):
    run = functools.partial(
        pl.kernel,
        mesh=_sc_mesh(),
        out_type=jax.ShapeDtypeStruct((NPAD, D), jnp.float32),
        scratch_types=[
            pltpu.VMEM((_TPW,), jnp.int32),
            pltpu.VMEM((_TPW,), jnp.int32),
            pltpu.VMEM((_TPW, D), jnp.float32),
            pltpu.SemaphoreType.DMA,
            pltpu.SemaphoreType.DMA,
        ],
    )(_sc_dispatch_body)
    return run(x, pos0.reshape(_GW, _TPW), pos1.reshape(_GW, _TPW))


def _sc_scatter_body(ys_hbm, didx_hbm, yu_hbm, idx_v, rows_v, sem):
    wid = lax.axis_index("s") * 2 + lax.axis_index("c")
    base = wid * _GPW
    for c in range(_GPW // _GC):
        o = base + c * _GC
        pltpu.sync_copy(didx_hbm.at[pl.ds(o, _GC)], idx_v)
        pltpu.sync_copy(ys_hbm.at[pl.ds(o, _GC)], rows_v)
        pltpu.async_copy(rows_v, yu_hbm.at[idx_v], sem).wait()


def _sc_scatter(ys, dest_slot, nb):
    del nb
    run = functools.partial(
        pl.kernel,
        mesh=_sc_mesh(),
        out_type=jax.ShapeDtypeStruct((NPAD, D), jnp.float32),
        scratch_types=[
            pltpu.VMEM((_GC,), jnp.int32),
            pltpu.VMEM((_GC, D), jnp.float32),
            pltpu.SemaphoreType.DMA,
        ],
    )(_sc_scatter_body)
    return run(ys, dest_slot)


def _ffn_body(be_ref, xb_ref, nb_ref, xs_ref, w1_ref, b1_ref, w2_ref, b2_ref,
              out_ref):
    bidx = pl.program_id(0)

    @pl.when(bidx < nb_ref[0])
    def _():
        h = jnp.dot(xs_ref[...], w1_ref[0],
                    preferred_element_type=jnp.float32) + b1_ref[0, 0][None, :]
        h = jnp.maximum(h, 0.0)
        y = jnp.dot(h, w2_ref[0],
                    preferred_element_type=jnp.float32) + b2_ref[0, 0][None, :]
        out_ref[...] = y


def _ffn(xs, w1, b1, w2, b2, be, nb):
    xb = jnp.minimum(jnp.arange(NB_MAX, dtype=jnp.int32), nb - 1)
    grid_spec = pltpu.PrefetchScalarGridSpec(
        num_scalar_prefetch=3,
        grid=(NB_MAX,),
        in_specs=[
            pl.BlockSpec((B, D), lambda b, be, xb, nbv: (xb[b], 0)),
            pl.BlockSpec((1, D, F), lambda b, be, xb, nbv: (be[b], 0, 0)),
            pl.BlockSpec((1, 1, F), lambda b, be, xb, nbv: (be[b], 0, 0)),
            pl.BlockSpec((1, F, D), lambda b, be, xb, nbv: (be[b], 0, 0)),
            pl.BlockSpec((1, 1, D), lambda b, be, xb, nbv: (be[b], 0, 0)),
        ],
        out_specs=pl.BlockSpec((B, D), lambda b, be, xb, nbv: (xb[b], 0)),
    )
    return pl.pallas_call(
        _ffn_body,
        grid_spec=grid_spec,
        out_shape=jax.ShapeDtypeStruct((NPAD, D), jnp.float32),
        compiler_params=pltpu.CompilerParams(
            dimension_semantics=("arbitrary",)),
    )(be, xb, nb.reshape(1), xs, w1, b1.reshape(NUM_E, 1, F), w2,
      b2.reshape(NUM_E, 1, D))


def _combine_body(yu_ref, g_ref, out_ref):
    y3 = yu_ref[...].reshape(yu_ref.shape[0] // K, K, D)
    out_ref[...] = (g_ref[:, 0:1] * y3[:, 0, :] + g_ref[:, 1:2] * y3[:, 1, :])


def _combine(yu, g):
    TB = 512
    return pl.pallas_call(
        _combine_body,
        grid=(T // TB,),
        in_specs=[
            pl.BlockSpec((TB * K, D), lambda i: (i, 0)),
            pl.BlockSpec((TB, K), lambda i: (i, 0)),
        ],
        out_specs=pl.BlockSpec((TB, D), lambda i: (i, 0)),
        out_shape=jax.ShapeDtypeStruct((T, D), jnp.float32),
    )(yu, g)


def kernel(moe_inp, gate_w, gate_b, w1, b1, w2, b2):
    topi, g, rank, counts = _gate(moe_inp, gate_w, gate_b)
    pos0, pos1, be, dest_slot, nb = _routing(topi, rank, counts[0])
    xs = _sc_dispatch(moe_inp, pos0, pos1)
    ys = _ffn(xs, w1, b1, w2, b2, be, nb)
    yu = _sc_scatter(ys, dest_slot, nb)
    return _combine(yu, g)
